# Initial kernel scaffold; baseline (speedup 1.0000x reference)
import jax
import jax.numpy as jnp
from jax.experimental import pallas as pl


def kernel(boxes, classes):
    raise NotImplementedError("write your pallas kernel here")


# speedup vs baseline: 1.0589x; 1.0589x over previous
import jax
import jax.numpy as jnp
from jax import lax
from jax.experimental import pallas as pl
from jax.experimental.pallas import tpu as pltpu

_IOU_T = 0.5
_SCORE_T = 0.5
_MAXDET = 100
_NEG_INF = float("-inf")


def _conf_body(x_ref, conf_ref, cid_ref):
    x = x_ref[0]
    m = jnp.max(x, axis=1)
    ii = lax.broadcasted_iota(jnp.int32, x.shape, 1)
    cid = jnp.min(jnp.where(x == m[:, None], ii, jnp.int32(2 ** 30)), axis=1)
    conf_ref[0, 0, :] = m
    cid_ref[0, 0, :] = cid


def _nms_body(byx_ref, conf_ref, cid_ref,
              oy0_ref, ox0_ref, oy1_ref, ox1_ref, oconf_ref, ocid_ref, onv_ref,
              work_ref, cy0_ref, cy1_ref, cx0_ref, cx1_ref, area_ref):
    ry0 = byx_ref[0]
    rx0 = byx_ref[1]
    ry1 = byx_ref[2]
    rx1 = byx_ref[3]
    cy0_ref[:, :] = jnp.minimum(ry0, ry1)
    cy1_ref[:, :] = jnp.maximum(ry0, ry1)
    cx0_ref[:, :] = jnp.minimum(rx0, rx1)
    cx1_ref[:, :] = jnp.maximum(rx0, rx1)
    area_ref[:, :] = (cy1_ref[:, :] - cy0_ref[:, :]) * (cx1_ref[:, :] - cx0_ref[:, :])
    conf2 = conf_ref[:, :]
    work_ref[:, :] = jnp.where(conf2 >= _SCORE_T, conf2, _NEG_INF)

    shape = conf2.shape
    acc_shape = (shape[0], 128)

    def body(i, carry):
        nval, ay0, ax0, ay1, ax1, aconf, acid = carry
        w = work_ref[:, :]
        m = jnp.max(w, axis=1, keepdims=True)
        ii = lax.broadcasted_iota(jnp.int32, shape, 1)
        idx = jnp.min(jnp.where(w == m, ii, jnp.int32(2 ** 30)), axis=1,
                      keepdims=True)
        selm = ii == idx

        def pickf(a):
            return jnp.sum(jnp.where(selm, a, jnp.float32(0.0)), axis=1,
                           keepdims=True)

        by0 = pickf(byx_ref[0])
        bx0 = pickf(byx_ref[1])
        by1 = pickf(byx_ref[2])
        bx1 = pickf(byx_ref[3])
        bconf = pickf(conf2)
        bcid = jnp.sum(jnp.where(selm, cid_ref[:, :], jnp.int32(0)), axis=1,
                       keepdims=True)

        valid = m > _NEG_INF

        ymin1 = jnp.minimum(by0, by1)
        ymax1 = jnp.maximum(by0, by1)
        xmin1 = jnp.minimum(bx0, bx1)
        xmax1 = jnp.maximum(bx0, bx1)
        inter_h = jnp.maximum(0.0, jnp.minimum(ymax1, cy1_ref[:, :])
                              - jnp.maximum(ymin1, cy0_ref[:, :]))
        inter_w = jnp.maximum(0.0, jnp.minimum(xmax1, cx1_ref[:, :])
                              - jnp.maximum(xmin1, cx0_ref[:, :]))
        inter = inter_h * inter_w
        area1 = (ymax1 - ymin1) * (xmax1 - xmin1)
        union = area1 + area_ref[:, :] - inter
        iou = jnp.where(union > 0, inter / union, jnp.float32(0.0))
        suppress = (iou > _IOU_T) & valid
        work_ref[:, :] = jnp.where(suppress | selm, _NEG_INF, w)

        colm = lax.broadcasted_iota(jnp.int32, acc_shape, 1) == i
        nval = nval + valid.astype(jnp.int32)
        ay0 = jnp.where(colm, by0, ay0)
        ax0 = jnp.where(colm, bx0, ax0)
        ay1 = jnp.where(colm, by1, ay1)
        ax1 = jnp.where(colm, bx1, ax1)
        aconf = jnp.where(colm, bconf, aconf)
        acid = jnp.where(colm, bcid, acid)
        return nval, ay0, ax0, ay1, ax1, aconf, acid

    zf = jnp.zeros(acc_shape, jnp.float32)
    zi = jnp.zeros(acc_shape, jnp.int32)
    init = (jnp.zeros((shape[0], 1), jnp.int32), zf, zf, zf, zf, zf, zi)
    nval, ay0, ax0, ay1, ax1, aconf, acid = lax.fori_loop(
        0, _MAXDET, body, init)
    oy0_ref[:, :] = ay0
    ox0_ref[:, :] = ax0
    oy1_ref[:, :] = ay1
    ox1_ref[:, :] = ax1
    oconf_ref[:, :] = aconf
    ocid_ref[:, :] = acid
    onv_ref[:, :] = jnp.broadcast_to(nval, acc_shape)


def _run(boxes, classes, interpret=False):
    B, N, _ = classes.shape
    CHUNK = 1000
    G = (B * N) // CHUNK
    cls_r = classes.reshape(G, CHUNK, 80)
    conf3, cid3 = pl.pallas_call(
        _conf_body,
        grid=(G,),
        in_specs=[pl.BlockSpec((1, CHUNK, 80), lambda i: (i, 0, 0))],
        out_specs=[pl.BlockSpec((1, 1, CHUNK), lambda i: (i, 0, 0)),
                   pl.BlockSpec((1, 1, CHUNK), lambda i: (i, 0, 0))],
        out_shape=[jax.ShapeDtypeStruct((G, 1, CHUNK), jnp.float32),
                   jax.ShapeDtypeStruct((G, 1, CHUNK), jnp.int32)],
        interpret=interpret,
    )(cls_r)

    NPAD = 20480
    pad = NPAD - N
    conf_p = jnp.pad(conf3.reshape(B, N), ((0, 0), (0, pad)),
                     constant_values=-1.0)
    cid_p = jnp.pad(cid3.reshape(B, N), ((0, 0), (0, pad)))
    byx = jnp.pad(boxes.transpose(2, 0, 1), ((0, 0), (0, 0), (0, pad)))

    outs = pl.pallas_call(
        _nms_body,
        out_shape=[jax.ShapeDtypeStruct((B, 128), jnp.float32)] * 5
        + [jax.ShapeDtypeStruct((B, 128), jnp.int32)] * 2,
        scratch_shapes=[pltpu.VMEM((B, NPAD), jnp.float32)] * 6,
        interpret=interpret,
    )(byx, conf_p, cid_p)
    oy0, ox0, oy1, ox1, oconf, ocid, onv = outs

    box_pred = jnp.stack(
        [oy0[:, :_MAXDET], ox0[:, :_MAXDET], oy1[:, :_MAXDET],
         ox1[:, :_MAXDET]], axis=-1)
    conf_pred = oconf[:, :_MAXDET]
    class_ids = ocid[:, :_MAXDET]
    valid_det = onv[:, 0]
    return box_pred, conf_pred, class_ids, valid_det


def kernel(boxes, classes):
    return _run(boxes, classes, interpret=False)


# speedup vs baseline: 1.3254x; 1.2516x over previous
import jax
import jax.numpy as jnp
from jax import lax
from jax.experimental import pallas as pl
from jax.experimental.pallas import tpu as pltpu
from jax.experimental.pallas import tpu_sc as plsc

_IOU_T = 0.5
_SCORE_T = 0.5
_MAXDET = 100
_NEG_INF = float("-inf")

_T0 = 0.999676
_CAP = 256
_QN = 5000
_QPAD = 5008
_NQ = 4
_NC = 2
_NS = 16


def _conf_body(x_ref, conf_ref, cid_ref):
    x = x_ref[0]
    m = jnp.max(x, axis=1)
    ii = lax.broadcasted_iota(jnp.int32, x.shape, 1)
    cid = jnp.min(jnp.where(x == m[:, None], ii, jnp.int32(2 ** 30)), axis=1)
    conf_ref[0, 0, :] = m
    cid_ref[0, 0, :] = cid


def _nms_body(byx_ref, conf_ref, cid_ref,
              oy0_ref, ox0_ref, oy1_ref, ox1_ref, oconf_ref, ocid_ref, onv_ref,
              work_ref, cy0_ref, cy1_ref, cx0_ref, cx1_ref, area_ref):
    ry0 = byx_ref[0]
    rx0 = byx_ref[1]
    ry1 = byx_ref[2]
    rx1 = byx_ref[3]
    cy0_ref[:, :] = jnp.minimum(ry0, ry1)
    cy1_ref[:, :] = jnp.maximum(ry0, ry1)
    cx0_ref[:, :] = jnp.minimum(rx0, rx1)
    cx1_ref[:, :] = jnp.maximum(rx0, rx1)
    area_ref[:, :] = (cy1_ref[:, :] - cy0_ref[:, :]) * (cx1_ref[:, :] - cx0_ref[:, :])
    conf2 = conf_ref[:, :]
    work_ref[:, :] = jnp.where(conf2 >= _SCORE_T, conf2, _NEG_INF)

    shape = conf2.shape
    acc_shape = (shape[0], 128)

    def body(i, carry):
        nval, ay0, ax0, ay1, ax1, aconf, acid = carry
        w = work_ref[:, :]
        m = jnp.max(w, axis=1, keepdims=True)
        ii = lax.broadcasted_iota(jnp.int32, shape, 1)
        idx = jnp.min(jnp.where(w == m, ii, jnp.int32(2 ** 30)), axis=1,
                      keepdims=True)
        selm = ii == idx

        def pickf(a):
            return jnp.sum(jnp.where(selm, a, jnp.float32(0.0)), axis=1,
                           keepdims=True)

        by0 = pickf(byx_ref[0])
        bx0 = pickf(byx_ref[1])
        by1 = pickf(byx_ref[2])
        bx1 = pickf(byx_ref[3])
        bconf = pickf(conf2)
        bcid = jnp.sum(jnp.where(selm, cid_ref[:, :], jnp.int32(0)), axis=1,
                       keepdims=True)

        valid = m > _NEG_INF

        ymin1 = jnp.minimum(by0, by1)
        ymax1 = jnp.maximum(by0, by1)
        xmin1 = jnp.minimum(bx0, bx1)
        xmax1 = jnp.maximum(bx0, bx1)
        inter_h = jnp.maximum(0.0, jnp.minimum(ymax1, cy1_ref[:, :])
                              - jnp.maximum(ymin1, cy0_ref[:, :]))
        inter_w = jnp.maximum(0.0, jnp.minimum(xmax1, cx1_ref[:, :])
                              - jnp.maximum(xmin1, cx0_ref[:, :]))
        inter = inter_h * inter_w
        area1 = (ymax1 - ymin1) * (xmax1 - xmin1)
        union = area1 + area_ref[:, :] - inter
        iou = jnp.where(union > 0, inter / union, jnp.float32(0.0))
        suppress = (iou > _IOU_T) & valid
        work_ref[:, :] = jnp.where(suppress | selm, _NEG_INF, w)

        colm = lax.broadcasted_iota(jnp.int32, acc_shape, 1) == i
        nval = nval + valid.astype(jnp.int32)
        ay0 = jnp.where(colm, by0, ay0)
        ax0 = jnp.where(colm, bx0, ax0)
        ay1 = jnp.where(colm, by1, ay1)
        ax1 = jnp.where(colm, bx1, ax1)
        aconf = jnp.where(colm, bconf, aconf)
        acid = jnp.where(colm, bcid, acid)
        return nval, ay0, ax0, ay1, ax1, aconf, acid

    zf = jnp.zeros(acc_shape, jnp.float32)
    zi = jnp.zeros(acc_shape, jnp.int32)
    init = (jnp.zeros((shape[0], 1), jnp.int32), zf, zf, zf, zf, zf, zi)
    nval, ay0, ax0, ay1, ax1, aconf, acid = lax.fori_loop(
        0, _MAXDET, body, init)
    oy0_ref[:, :] = ay0
    ox0_ref[:, :] = ax0
    oy1_ref[:, :] = ay1
    ox1_ref[:, :] = ax1
    oconf_ref[:, :] = aconf
    ocid_ref[:, :] = acid
    onv_ref[:, :] = jnp.broadcast_to(nval, acc_shape)


def _nms_call(byx, conf, cid, interpret=False):
    B = conf.shape[0]
    N = conf.shape[1]
    outs = pl.pallas_call(
        _nms_body,
        out_shape=[jax.ShapeDtypeStruct((B, 128), jnp.float32)] * 5
        + [jax.ShapeDtypeStruct((B, 128), jnp.int32)] * 2,
        scratch_shapes=[pltpu.VMEM((B, N), jnp.float32)] * 6,
        interpret=interpret,
    )(byx, conf, cid)
    return outs


def _compact_body(byx_ref, conf_ref, cid_ref,
                  py0_ref, px0_ref, py1_ref, px1_ref, pconf_ref, pcid_ref,
                  pcnt_ref,
                  y0_b, x0_b, y1_b, x1_b, conf_b, cid_b,
                  idx_b, oy0_b, ox0_b, oy1_b, ox1_b, oconf_b, ocid_b, ocnt_b):
    wid = lax.axis_index("s") * _NC + lax.axis_index("c")
    b = wid // _NQ
    q = wid % _NQ
    base = q * _QN

    pltpu.sync_copy(conf_ref.at[b, pl.ds(base, _QN)], conf_b.at[pl.ds(0, _QN)])
    pltpu.sync_copy(byx_ref.at[0, b, pl.ds(base, _QN)], y0_b.at[pl.ds(0, _QN)])
    pltpu.sync_copy(byx_ref.at[1, b, pl.ds(base, _QN)], x0_b.at[pl.ds(0, _QN)])
    pltpu.sync_copy(byx_ref.at[2, b, pl.ds(base, _QN)], y1_b.at[pl.ds(0, _QN)])
    pltpu.sync_copy(byx_ref.at[3, b, pl.ds(base, _QN)], x1_b.at[pl.ds(0, _QN)])
    pltpu.sync_copy(cid_ref.at[b, pl.ds(base, _QN)], cid_b.at[pl.ds(0, _QN)])

    iota16 = lax.iota(jnp.int32, 16)

    def scan_step(i, cnt):
        v = conf_b[pl.ds(i * 16, 16)]
        inb = (iota16 + i * 16) < _QN
        m = (v >= _T0) & inb
        mi = m.astype(jnp.int32)
        pos = cnt + plsc.cumsum(mi) - 1
        okm = m & (pos < _CAP)
        plsc.store_scatter(idx_b, [pos], iota16 + i * 16, mask=okm)
        return cnt + plsc.all_reduce_population_count(m)

    nsteps = _QPAD // 16
    cnt = lax.fori_loop(0, nsteps, scan_step,
                        jnp.zeros((16,), jnp.int32))
    ocnt_b[...] = cnt

    def gather_step(j, _):
        raw = idx_b[pl.ds(j * 16, 16)]
        valid = (iota16 + j * 16) < cnt
        idxs = jnp.where(valid, raw, 0)
        sl = pl.ds(j * 16, 16)
        oconf_b[sl] = jnp.where(valid, plsc.load_gather(conf_b, [idxs]),
                                _NEG_INF)
        oy0_b[sl] = jnp.where(valid, plsc.load_gather(y0_b, [idxs]), 0.0)
        ox0_b[sl] = jnp.where(valid, plsc.load_gather(x0_b, [idxs]), 0.0)
        oy1_b[sl] = jnp.where(valid, plsc.load_gather(y1_b, [idxs]), 0.0)
        ox1_b[sl] = jnp.where(valid, plsc.load_gather(x1_b, [idxs]), 0.0)
        ocid_b[sl] = jnp.where(valid, plsc.load_gather(cid_b, [idxs]),
                               jnp.int32(0))
        return 0

    lax.fori_loop(0, _CAP // 16, gather_step, 0)

    pltpu.sync_copy(oy0_b, py0_ref.at[b, q])
    pltpu.sync_copy(ox0_b, px0_ref.at[b, q])
    pltpu.sync_copy(oy1_b, py1_ref.at[b, q])
    pltpu.sync_copy(ox1_b, px1_ref.at[b, q])
    pltpu.sync_copy(oconf_b, pconf_ref.at[b, q])
    pltpu.sync_copy(ocid_b, pcid_ref.at[b, q])
    pltpu.sync_copy(ocnt_b, pcnt_ref.at[b, q])


def _compact_call(byx, conf, cid):
    B = conf.shape[0]
    f32 = jnp.float32
    i32 = jnp.int32
    mesh = plsc.VectorSubcoreMesh(core_axis_name="c", subcore_axis_name="s",
                                  num_cores=_NC, num_subcores=_NS)
    out_type = (
        [jax.ShapeDtypeStruct((B, _NQ, _CAP), f32)] * 5
        + [jax.ShapeDtypeStruct((B, _NQ, _CAP), i32)]
        + [jax.ShapeDtypeStruct((B, _NQ, 16), i32)]
    )
    scratch = (
        [pltpu.VMEM((_QPAD,), f32)] * 5
        + [pltpu.VMEM((_QPAD,), i32)]
        + [pltpu.VMEM((_CAP,), i32)]
        + [pltpu.VMEM((_CAP,), f32)] * 5
        + [pltpu.VMEM((_CAP,), i32)]
        + [pltpu.VMEM((16,), i32)]
    )
    fn = pl.kernel(_compact_body, out_type=out_type, mesh=mesh,
                   scratch_types=scratch,
                   compiler_params=pltpu.CompilerParams(
                       use_tc_tiling_on_sc=False,
                       needs_layout_passes=False))
    return fn(byx, conf, cid)


def kernel(boxes, classes):
    B, N, _ = classes.shape
    CHUNK = 1000
    G = (B * N) // CHUNK
    cls_r = classes.reshape(G, CHUNK, 80)
    conf3, cid3 = pl.pallas_call(
        _conf_body,
        grid=(G,),
        in_specs=[pl.BlockSpec((1, CHUNK, 80), lambda i: (i, 0, 0))],
        out_specs=[pl.BlockSpec((1, 1, CHUNK), lambda i: (i, 0, 0)),
                   pl.BlockSpec((1, 1, CHUNK), lambda i: (i, 0, 0))],
        out_shape=[jax.ShapeDtypeStruct((G, 1, CHUNK), jnp.float32),
                   jax.ShapeDtypeStruct((G, 1, CHUNK), jnp.int32)],
    )(cls_r)

    NPAD = 20480
    pad = NPAD - N
    conf2 = conf3.reshape(B, N)
    cid2 = cid3.reshape(B, N)
    conf_p = jnp.pad(conf2, ((0, 0), (0, pad)), constant_values=-1.0)
    cid_p = jnp.pad(cid2, ((0, 0), (0, pad)))
    byx = jnp.pad(boxes.transpose(2, 0, 1), ((0, 0), (0, 0), (0, pad)))

    py0, px0, py1, px1, pconf, pcid, pcnt = _compact_call(byx, conf2, cid2)
    PW = _NQ * _CAP
    pool_byx = jnp.stack([py0.reshape(B, PW), px0.reshape(B, PW),
                          py1.reshape(B, PW), px1.reshape(B, PW)], axis=0)
    fast = _nms_call(pool_byx, pconf.reshape(B, PW), pcid.reshape(B, PW))

    overflow = jnp.any(pcnt[:, :, 0] > _CAP)
    short = jnp.any(fast[6][:, 0] < _MAXDET)

    outs = lax.cond(overflow | short,
                    lambda: tuple(_nms_call(byx, conf_p, cid_p)),
                    lambda: tuple(fast))
    oy0, ox0, oy1, ox1, oconf, ocid, onv = outs

    box_pred = jnp.stack(
        [oy0[:, :_MAXDET], ox0[:, :_MAXDET], oy1[:, :_MAXDET],
         ox1[:, :_MAXDET]], axis=-1)
    conf_pred = oconf[:, :_MAXDET]
    class_ids = ocid[:, :_MAXDET]
    valid_det = onv[:, 0]
    return box_pred, conf_pred, class_ids, valid_det
